# baseline (speedup 1.0000x reference)
import jax
import jax.numpy as jnp
from jax import lax
from jax.experimental import pallas as pl
from jax.experimental.pallas import tpu as pltpu
from jax.experimental.pallas import tpu_sc as plsc

N = 10000
E = 160000
D = 256
R = 8
NB = 4
L = 4
OUT = 128
RN = R * N

NSC = 2
NTILE = 16
LANE = 16
HALF = D // NSC
EPT = E // NTILE
CH = 64
NCHUNK = 160
NBANK = 4
NQ = NCHUNK // NBANK
EPT_PAD = NCHUNK * CH
NACC = N + LANE
ROWS_T = 624
ROWS_LAST = N - (NTILE - 1) * ROWS_T
BN_ROWS = 1000
NI = N // BN_ROWS


def _sc_body(hw_hbm, ed_hbm, out_hbm,
             eb0, eb1, eb2, eb3, gbuf0, gbuf1, gbuf2, gbuf3, zbuf, acc,
             gsem0, gsem1, gsem2, gsem3, ssem0, ssem1, ssem2, ssem3):
    sc = lax.axis_index("c")
    t = lax.axis_index("s")
    base = t * ROWS_T
    ebs = (eb0, eb1, eb2, eb3)
    gbufs = (gbuf0, gbuf1, gbuf2, gbuf3)
    gsems = (gsem0, gsem1, gsem2, gsem3)
    ssems = (ssem0, ssem1, ssem2, ssem3)

    goff = sc * RN

    def _bld(eb):
        def _bi(c, carry):
            sl = pl.ds(c * LANE, LANE)
            eb[0, sl] = eb[2, sl] * N + eb[0, sl] + goff
            return carry
        lax.fori_loop(0, CH // LANE, _bi, 0)

    def _zb(i, carry):
        def _zbi(c, carry2):
            zbuf[i, pl.ds(c * LANE, LANE)] = jnp.zeros((LANE,), jnp.float32)
            return carry2
        return lax.fori_loop(0, HALF // LANE, _zbi, carry)

    lax.fori_loop(0, LANE, _zb, 0)

    nz = lax.select(t == NTILE - 1, ROWS_LAST // LANE, ROWS_T // LANE)

    def _z(i, carry):
        pltpu.sync_copy(zbuf, acc.at[pl.ds(base + i * LANE, LANE)])
        return carry

    lax.fori_loop(0, nz, _z, 0)
    plsc.subcore_barrier()

    for b in range(NBANK):
        pltpu.sync_copy(ed_hbm.at[t, b], ebs[b])
        _bld(ebs[b])
        pltpu.async_copy(hw_hbm.at[ebs[b].at[0]], gbufs[b], gsems[b])

    def _quad(q, carry):
        j = NBANK * q
        for b in range(NBANK):
            pltpu.make_async_copy(hw_hbm.at[ebs[b].at[0]], gbufs[b],
                                  gsems[b]).wait()
            pltpu.async_copy(gbufs[b], acc.at[ebs[b].at[1]], ssems[b],
                             add=True)

            @pl.when(j + b + NBANK < NCHUNK)
            def _(b=b):
                pltpu.make_async_copy(gbufs[b], acc.at[ebs[b].at[1]],
                                      ssems[b]).wait()
                pltpu.sync_copy(ed_hbm.at[t, j + b + NBANK], ebs[b])
                _bld(ebs[b])
                pltpu.async_copy(hw_hbm.at[ebs[b].at[0]], gbufs[b], gsems[b])
        return carry

    lax.fori_loop(0, NQ, _quad, 0)
    for b in range(NBANK):
        pltpu.make_async_copy(gbufs[b], acc.at[ebs[b].at[1]],
                              ssems[b]).wait()
    plsc.subcore_barrier()

    csl = pl.ds(sc * HALF, HALF)

    @pl.when(t < NTILE - 1)
    def _():
        pltpu.sync_copy(acc.at[pl.ds(base, ROWS_T)],
                        out_hbm.at[pl.ds(base, ROWS_T), csl])

    @pl.when(t == NTILE - 1)
    def _():
        pltpu.sync_copy(acc.at[pl.ds(base, ROWS_LAST)],
                        out_hbm.at[pl.ds(base, ROWS_LAST), csl])


def _sc_aggregate(hw_flat, edata):
    mesh = plsc.VectorSubcoreMesh(core_axis_name="c", subcore_axis_name="s")
    f = pl.kernel(
        _sc_body,
        out_type=jax.ShapeDtypeStruct((N, D), jnp.float32),
        mesh=mesh,
        scratch_types=(
            [pltpu.VMEM((3, CH), jnp.int32) for _ in range(NBANK)]
            + [pltpu.VMEM((CH, HALF), jnp.float32) for _ in range(NBANK)]
            + [pltpu.VMEM((LANE, HALF), jnp.float32),
               pltpu.VMEM_SHARED((NACC, HALF), jnp.float32)]
            + [pltpu.SemaphoreType.DMA for _ in range(2 * NBANK)]
        ),
    )
    return f(hw_flat, edata)


def _wfull_body(coef_ref, basis_ref, out_ref):
    acc = jnp.zeros((D, D), jnp.float32)
    for b in range(NB):
        acc = acc + coef_ref[0, 0, b] * basis_ref[b]
    out_ref[0] = acc


def _wfull(coef_l3, basis_l):
    return pl.pallas_call(
        _wfull_body,
        grid=(R,),
        in_specs=[
            pl.BlockSpec((1, 1, NB), lambda r: (r, 0, 0)),
            pl.BlockSpec((NB, D, D), lambda r: (0, 0, 0)),
        ],
        out_specs=pl.BlockSpec((1, D, D), lambda r: (r, 0, 0)),
        out_shape=jax.ShapeDtypeStruct((R, D, D), jnp.float32),
    )(coef_l3, basis_l)


def _hw_body(h_ref, w_ref, out_ref):
    out_ref[0, 0] = jnp.dot(h_ref[...], w_ref[0],
                            preferred_element_type=jnp.float32)


def _hw(h, wfull):
    return pl.pallas_call(
        _hw_body,
        grid=(NI, R, NSC),
        in_specs=[
            pl.BlockSpec((BN_ROWS, D), lambda ii, r, c: (ii, 0)),
            pl.BlockSpec((1, D, HALF), lambda ii, r, c: (r, 0, c)),
        ],
        out_specs=pl.BlockSpec((1, 1, BN_ROWS, HALF),
                               lambda ii, r, c: (c, r, ii, 0)),
        out_shape=jax.ShapeDtypeStruct((NSC, R, N, HALF), jnp.float32),
    )(h, wfull)


def _mlp1_body(h_ref, agg_ref, w_ref, y_ref, st_ref):
    i = pl.program_id(0)
    z = h_ref[...] + agg_ref[...]
    y = jnp.dot(z, w_ref[...], preferred_element_type=jnp.float32)
    y_ref[...] = y
    s1 = jnp.sum(y, axis=0, keepdims=True)
    s2 = jnp.sum(y * y, axis=0, keepdims=True)
    st = jnp.concatenate([s1, s2, jnp.zeros((6, D), jnp.float32)], axis=0)

    @pl.when(i == 0)
    def _():
        st_ref[...] = jnp.zeros_like(st_ref)

    st_ref[...] = st_ref[...] + st


def _mlp1(h, agg, w):
    return pl.pallas_call(
        _mlp1_body,
        grid=(NI,),
        in_specs=[
            pl.BlockSpec((BN_ROWS, D), lambda i: (i, 0)),
            pl.BlockSpec((BN_ROWS, D), lambda i: (i, 0)),
            pl.BlockSpec((D, D), lambda i: (0, 0)),
        ],
        out_specs=[
            pl.BlockSpec((BN_ROWS, D), lambda i: (i, 0)),
            pl.BlockSpec((8, D), lambda i: (0, 0)),
        ],
        out_shape=[
            jax.ShapeDtypeStruct((N, D), jnp.float32),
            jax.ShapeDtypeStruct((8, D), jnp.float32),
        ],
    )(h, agg, w)


def _bn_mm_stats_body(t_ref, st_in_ref, g_ref, b_ref, w_ref, y_ref, st_ref):
    i = pl.program_id(0)
    mean = st_in_ref[0:1, :] * (1.0 / N)
    ex2 = st_in_ref[1:2, :] * (1.0 / N)
    var = ex2 - mean * mean
    inv = g_ref[...] * lax.rsqrt(var + 1e-5)
    a = jnp.maximum((t_ref[...] - mean) * inv + b_ref[...], 0.0)
    y = jnp.dot(a, w_ref[...], preferred_element_type=jnp.float32)
    y_ref[...] = y
    s1 = jnp.sum(y, axis=0, keepdims=True)
    s2 = jnp.sum(y * y, axis=0, keepdims=True)
    st = jnp.concatenate([s1, s2, jnp.zeros((6, D), jnp.float32)], axis=0)

    @pl.when(i == 0)
    def _():
        st_ref[...] = jnp.zeros_like(st_ref)

    st_ref[...] = st_ref[...] + st


def _bn_mm_stats(t, st1, g, b, w):
    return pl.pallas_call(
        _bn_mm_stats_body,
        grid=(NI,),
        in_specs=[
            pl.BlockSpec((BN_ROWS, D), lambda i: (i, 0)),
            pl.BlockSpec((8, D), lambda i: (0, 0)),
            pl.BlockSpec((1, D), lambda i: (0, 0)),
            pl.BlockSpec((1, D), lambda i: (0, 0)),
            pl.BlockSpec((D, D), lambda i: (0, 0)),
        ],
        out_specs=[
            pl.BlockSpec((BN_ROWS, D), lambda i: (i, 0)),
            pl.BlockSpec((8, D), lambda i: (0, 0)),
        ],
        out_shape=[
            jax.ShapeDtypeStruct((N, D), jnp.float32),
            jax.ShapeDtypeStruct((8, D), jnp.float32),
        ],
    )(t, st1, g, b, w)


def _bn_final_body(u_ref, st_ref, g_ref, b_ref, h_ref, p_ref):
    ii = pl.program_id(0)
    mean = st_ref[0:1, :] * (1.0 / N)
    ex2 = st_ref[1:2, :] * (1.0 / N)
    var = ex2 - mean * mean
    inv = g_ref[...] * lax.rsqrt(var + 1e-5)
    h = jnp.maximum((u_ref[...] - mean) * inv + b_ref[...], 0.0)
    h_ref[...] = h
    ps = jnp.concatenate(
        [jnp.sum(h, axis=0, keepdims=True), jnp.zeros((7, D), jnp.float32)],
        axis=0)

    @pl.when(ii == 0)
    def _():
        p_ref[...] = jnp.zeros_like(p_ref)

    p_ref[...] = p_ref[...] + ps


def _bn_final(u, st2, g, b):
    return pl.pallas_call(
        _bn_final_body,
        grid=(NI,),
        in_specs=[
            pl.BlockSpec((BN_ROWS, D), lambda ii: (ii, 0)),
            pl.BlockSpec((8, D), lambda ii: (0, 0)),
            pl.BlockSpec((1, D), lambda ii: (0, 0)),
            pl.BlockSpec((1, D), lambda ii: (0, 0)),
        ],
        out_specs=[
            pl.BlockSpec((BN_ROWS, D), lambda ii: (ii, 0)),
            pl.BlockSpec((8, D), lambda ii: (0, 0)),
        ],
        out_shape=[
            jax.ShapeDtypeStruct((N, D), jnp.float32),
            jax.ShapeDtypeStruct((8, D), jnp.float32),
        ],
    )(u, st2, g, b)


def _pool_body(x_ref, p_ref):
    ii = pl.program_id(0)
    ps = jnp.concatenate(
        [jnp.sum(x_ref[...], axis=0, keepdims=True),
         jnp.zeros((7, D), jnp.float32)], axis=0)

    @pl.when(ii == 0)
    def _():
        p_ref[...] = jnp.zeros_like(p_ref)

    p_ref[...] = p_ref[...] + ps


def _pool(x):
    return pl.pallas_call(
        _pool_body,
        grid=(NI,),
        in_specs=[pl.BlockSpec((BN_ROWS, D), lambda ii: (ii, 0))],
        out_specs=pl.BlockSpec((8, D), lambda ii: (0, 0)),
        out_shape=jax.ShapeDtypeStruct((8, D), jnp.float32),
    )(x)


def _final_body(p_ref, wp_ref, bp_ref, out_ref):
    acc = jnp.zeros((1, OUT), jnp.float32)
    for i in range(L + 1):
        acc = acc + jnp.dot(p_ref[i:i + 1, :], wp_ref[i],
                            preferred_element_type=jnp.float32)
    acc = acc + jnp.sum(bp_ref[:, 0, :], axis=0, keepdims=True)
    out_ref[...] = acc


def _final(pall, wp, bp3):
    return pl.pallas_call(
        _final_body,
        grid=(1,),
        in_specs=[
            pl.BlockSpec((8, D), lambda i: (0, 0)),
            pl.BlockSpec((L + 1, D, OUT), lambda i: (0, 0, 0)),
            pl.BlockSpec((L + 1, 1, OUT), lambda i: (0, 0, 0)),
        ],
        out_specs=pl.BlockSpec((1, OUT), lambda i: (0, 0)),
        out_shape=jax.ShapeDtypeStruct((1, OUT), jnp.float32),
    )(pall, wp, bp3)


def kernel(x, edge_index, rel_type, basis, coef, W1, bn1_g, bn1_b,
           W2, bn2_g, bn2_b, Wp, bp):
    src = edge_index[0].astype(jnp.int32)
    dst = edge_index[1].astype(jnp.int32)
    rel = rel_type.astype(jnp.int32)

    pad = EPT_PAD - EPT
    src_t = jnp.pad(src.reshape(NTILE, EPT), ((0, 0), (0, pad)),
                    constant_values=0).reshape(NTILE, NCHUNK, CH)
    dst_t = jnp.pad(dst.reshape(NTILE, EPT), ((0, 0), (0, pad)),
                    constant_values=N).reshape(NTILE, NCHUNK, CH)
    rel_t = jnp.pad(rel.reshape(NTILE, EPT), ((0, 0), (0, pad)),
                    constant_values=0).reshape(NTILE, NCHUNK, CH)
    edata = jnp.stack([src_t, dst_t, rel_t], axis=2)

    pools = [_pool(x)]
    h = x
    for l in range(L):
        wfull = _wfull(coef[l].reshape(R, 1, NB), basis[l])
        hw = _hw(h, wfull)
        agg = _sc_aggregate(hw.reshape(NSC * RN, HALF), edata)
        t, st1 = _mlp1(h, agg, W1[l])
        u, st2 = _bn_mm_stats(t, st1, bn1_g[l].reshape(1, D),
                              bn1_b[l].reshape(1, D), W2[l])
        h, ph = _bn_final(u, st2, bn2_g[l].reshape(1, D),
                          bn2_b[l].reshape(1, D))
        pools.append(ph)

    pall = jnp.concatenate([p[0:1] for p in pools]
                           + [jnp.zeros((8 - (L + 1), D), jnp.float32)],
                           axis=0)
    return _final(pall, Wp, bp.reshape(L + 1, 1, OUT))


# speedup vs baseline: 1.0166x; 1.0166x over previous
import jax
import jax.numpy as jnp
from jax import lax
from jax.experimental import pallas as pl
from jax.experimental.pallas import tpu as pltpu
from jax.experimental.pallas import tpu_sc as plsc

N = 10000
E = 160000
D = 256
R = 8
NB = 4
L = 4
OUT = 128
RN = R * N

NSC = 2
NTILE = 16
LANE = 16
HALF = D // NSC
EPT = E // NTILE
CH = 128
NCHUNK = 81
NBANK = 3
NQ = NCHUNK // NBANK
EPT_PAD = NCHUNK * CH
NACC = N + LANE
ROWS_T = 624
ROWS_LAST = N - (NTILE - 1) * ROWS_T
BN_ROWS = 1000
NI = N // BN_ROWS


def _sc_body(hw0_hbm, hw1_hbm, zrows_hbm, ed_hbm, out_hbm,
             eb0, eb1, eb2, gbuf0, gbuf1, gbuf2, acc,
             gsem0, gsem1, gsem2, ssem0, ssem1, ssem2):
    sc = lax.axis_index("c")
    t = lax.axis_index("s")
    base = t * ROWS_T
    ebs = (eb0, eb1, eb2)
    gbufs = (gbuf0, gbuf1, gbuf2)
    gsems = (gsem0, gsem1, gsem2)
    ssems = (ssem0, ssem1, ssem2)

    @pl.when(t < NTILE - 1)
    def _():
        pltpu.sync_copy(zrows_hbm.at[pl.ds(0, ROWS_T)],
                        acc.at[pl.ds(base, ROWS_T)])

    @pl.when(t == NTILE - 1)
    def _():
        pltpu.sync_copy(zrows_hbm, acc.at[pl.ds(base, ROWS_LAST)])

    plsc.subcore_barrier()

    def _run(tab):
        for b in range(NBANK):
            pltpu.sync_copy(ed_hbm.at[t, b], ebs[b])
            pltpu.async_copy(tab.at[ebs[b].at[0]], gbufs[b], gsems[b])

        def _q(q, carry):
            j = NBANK * q
            for b in range(NBANK):
                pltpu.make_async_copy(tab.at[ebs[b].at[0]], gbufs[b],
                                      gsems[b]).wait()
                pltpu.async_copy(gbufs[b], acc.at[ebs[b].at[1]], ssems[b],
                                 add=True)

                @pl.when(j + b + NBANK < NCHUNK)
                def _(b=b):
                    pltpu.make_async_copy(gbufs[b], acc.at[ebs[b].at[1]],
                                          ssems[b]).wait()
                    pltpu.sync_copy(ed_hbm.at[t, j + b + NBANK], ebs[b])
                    pltpu.async_copy(tab.at[ebs[b].at[0]], gbufs[b],
                                     gsems[b])
            return carry

        lax.fori_loop(0, NQ, _q, 0)
        for b in range(NBANK):
            pltpu.make_async_copy(gbufs[b], acc.at[ebs[b].at[1]],
                                  ssems[b]).wait()

    @pl.when(sc == 0)
    def _():
        _run(hw0_hbm)

    @pl.when(sc == 1)
    def _():
        _run(hw1_hbm)

    plsc.subcore_barrier()

    csl = pl.ds(sc * HALF, HALF)

    @pl.when(t < NTILE - 1)
    def _():
        pltpu.sync_copy(acc.at[pl.ds(base, ROWS_T)],
                        out_hbm.at[pl.ds(base, ROWS_T), csl])

    @pl.when(t == NTILE - 1)
    def _():
        pltpu.sync_copy(acc.at[pl.ds(base, ROWS_LAST)],
                        out_hbm.at[pl.ds(base, ROWS_LAST), csl])


def _sc_aggregate(hw0, hw1, zrows, edata):
    mesh = plsc.VectorSubcoreMesh(core_axis_name="c", subcore_axis_name="s")
    f = pl.kernel(
        _sc_body,
        out_type=jax.ShapeDtypeStruct((N, D), jnp.float32),
        mesh=mesh,
        scratch_types=(
            [pltpu.VMEM((2, CH), jnp.int32) for _ in range(NBANK)]
            + [pltpu.VMEM((CH, HALF), jnp.float32) for _ in range(NBANK)]
            + [pltpu.VMEM_SHARED((NACC, HALF), jnp.float32)]
            + [pltpu.SemaphoreType.DMA for _ in range(2 * NBANK)]
        ),
    )
    return f(hw0, hw1, zrows, edata)


def _wfull_body(coef_ref, basis_ref, out_ref):
    acc = jnp.zeros((D, D), jnp.float32)
    for b in range(NB):
        acc = acc + coef_ref[0, 0, b] * basis_ref[b]
    out_ref[0] = acc


def _wfull(coef_l3, basis_l):
    return pl.pallas_call(
        _wfull_body,
        grid=(R,),
        in_specs=[
            pl.BlockSpec((1, 1, NB), lambda r: (r, 0, 0)),
            pl.BlockSpec((NB, D, D), lambda r: (0, 0, 0)),
        ],
        out_specs=pl.BlockSpec((1, D, D), lambda r: (r, 0, 0)),
        out_shape=jax.ShapeDtypeStruct((R, D, D), jnp.float32),
    )(coef_l3, basis_l)


def _hw_body(h_ref, w_ref, out0_ref, out1_ref):
    y = jnp.dot(h_ref[...], w_ref[0], preferred_element_type=jnp.float32)
    out0_ref[0] = y[:, :HALF]
    out1_ref[0] = y[:, HALF:]


def _hw(h, wfull):
    return pl.pallas_call(
        _hw_body,
        grid=(NI, R),
        in_specs=[
            pl.BlockSpec((BN_ROWS, D), lambda ii, r: (ii, 0)),
            pl.BlockSpec((1, D, D), lambda ii, r: (r, 0, 0)),
        ],
        out_specs=[
            pl.BlockSpec((1, BN_ROWS, HALF), lambda ii, r: (r, ii, 0)),
            pl.BlockSpec((1, BN_ROWS, HALF), lambda ii, r: (r, ii, 0)),
        ],
        out_shape=[
            jax.ShapeDtypeStruct((R, N, HALF), jnp.float32),
            jax.ShapeDtypeStruct((R, N, HALF), jnp.float32),
        ],
    )(h, wfull)


def _mlp1_body(h_ref, agg_ref, w_ref, y_ref, st_ref):
    i = pl.program_id(0)
    z = h_ref[...] + agg_ref[...]
    y = jnp.dot(z, w_ref[...], preferred_element_type=jnp.float32)
    y_ref[...] = y
    s1 = jnp.sum(y, axis=0, keepdims=True)
    s2 = jnp.sum(y * y, axis=0, keepdims=True)
    st = jnp.concatenate([s1, s2, jnp.zeros((6, D), jnp.float32)], axis=0)

    @pl.when(i == 0)
    def _():
        st_ref[...] = jnp.zeros_like(st_ref)

    st_ref[...] = st_ref[...] + st


def _mlp1(h, agg, w):
    return pl.pallas_call(
        _mlp1_body,
        grid=(NI,),
        in_specs=[
            pl.BlockSpec((BN_ROWS, D), lambda i: (i, 0)),
            pl.BlockSpec((BN_ROWS, D), lambda i: (i, 0)),
            pl.BlockSpec((D, D), lambda i: (0, 0)),
        ],
        out_specs=[
            pl.BlockSpec((BN_ROWS, D), lambda i: (i, 0)),
            pl.BlockSpec((8, D), lambda i: (0, 0)),
        ],
        out_shape=[
            jax.ShapeDtypeStruct((N, D), jnp.float32),
            jax.ShapeDtypeStruct((8, D), jnp.float32),
        ],
    )(h, agg, w)


def _bn_mm_stats_body(t_ref, st_in_ref, g_ref, b_ref, w_ref, y_ref, st_ref):
    i = pl.program_id(0)
    mean = st_in_ref[0:1, :] * (1.0 / N)
    ex2 = st_in_ref[1:2, :] * (1.0 / N)
    var = ex2 - mean * mean
    inv = g_ref[...] * lax.rsqrt(var + 1e-5)
    a = jnp.maximum((t_ref[...] - mean) * inv + b_ref[...], 0.0)
    y = jnp.dot(a, w_ref[...], preferred_element_type=jnp.float32)
    y_ref[...] = y
    s1 = jnp.sum(y, axis=0, keepdims=True)
    s2 = jnp.sum(y * y, axis=0, keepdims=True)
    st = jnp.concatenate([s1, s2, jnp.zeros((6, D), jnp.float32)], axis=0)

    @pl.when(i == 0)
    def _():
        st_ref[...] = jnp.zeros_like(st_ref)

    st_ref[...] = st_ref[...] + st


def _bn_mm_stats(t, st1, g, b, w):
    return pl.pallas_call(
        _bn_mm_stats_body,
        grid=(NI,),
        in_specs=[
            pl.BlockSpec((BN_ROWS, D), lambda i: (i, 0)),
            pl.BlockSpec((8, D), lambda i: (0, 0)),
            pl.BlockSpec((1, D), lambda i: (0, 0)),
            pl.BlockSpec((1, D), lambda i: (0, 0)),
            pl.BlockSpec((D, D), lambda i: (0, 0)),
        ],
        out_specs=[
            pl.BlockSpec((BN_ROWS, D), lambda i: (i, 0)),
            pl.BlockSpec((8, D), lambda i: (0, 0)),
        ],
        out_shape=[
            jax.ShapeDtypeStruct((N, D), jnp.float32),
            jax.ShapeDtypeStruct((8, D), jnp.float32),
        ],
    )(t, st1, g, b, w)


def _bn_final_body(u_ref, st_ref, g_ref, b_ref, h_ref, p_ref):
    ii = pl.program_id(0)
    mean = st_ref[0:1, :] * (1.0 / N)
    ex2 = st_ref[1:2, :] * (1.0 / N)
    var = ex2 - mean * mean
    inv = g_ref[...] * lax.rsqrt(var + 1e-5)
    h = jnp.maximum((u_ref[...] - mean) * inv + b_ref[...], 0.0)
    h_ref[...] = h
    ps = jnp.concatenate(
        [jnp.sum(h, axis=0, keepdims=True), jnp.zeros((7, D), jnp.float32)],
        axis=0)

    @pl.when(ii == 0)
    def _():
        p_ref[...] = jnp.zeros_like(p_ref)

    p_ref[...] = p_ref[...] + ps


def _bn_final(u, st2, g, b):
    return pl.pallas_call(
        _bn_final_body,
        grid=(NI,),
        in_specs=[
            pl.BlockSpec((BN_ROWS, D), lambda ii: (ii, 0)),
            pl.BlockSpec((8, D), lambda ii: (0, 0)),
            pl.BlockSpec((1, D), lambda ii: (0, 0)),
            pl.BlockSpec((1, D), lambda ii: (0, 0)),
        ],
        out_specs=[
            pl.BlockSpec((BN_ROWS, D), lambda ii: (ii, 0)),
            pl.BlockSpec((8, D), lambda ii: (0, 0)),
        ],
        out_shape=[
            jax.ShapeDtypeStruct((N, D), jnp.float32),
            jax.ShapeDtypeStruct((8, D), jnp.float32),
        ],
    )(u, st2, g, b)


def _pool_body(x_ref, p_ref):
    ii = pl.program_id(0)
    ps = jnp.concatenate(
        [jnp.sum(x_ref[...], axis=0, keepdims=True),
         jnp.zeros((7, D), jnp.float32)], axis=0)

    @pl.when(ii == 0)
    def _():
        p_ref[...] = jnp.zeros_like(p_ref)

    p_ref[...] = p_ref[...] + ps


def _pool(x):
    return pl.pallas_call(
        _pool_body,
        grid=(NI,),
        in_specs=[pl.BlockSpec((BN_ROWS, D), lambda ii: (ii, 0))],
        out_specs=pl.BlockSpec((8, D), lambda ii: (0, 0)),
        out_shape=jax.ShapeDtypeStruct((8, D), jnp.float32),
    )(x)


def _final_body(p_ref, wp_ref, bp_ref, out_ref):
    acc = jnp.zeros((1, OUT), jnp.float32)
    for i in range(L + 1):
        acc = acc + jnp.dot(p_ref[i:i + 1, :], wp_ref[i],
                            preferred_element_type=jnp.float32)
    acc = acc + jnp.sum(bp_ref[:, 0, :], axis=0, keepdims=True)
    out_ref[...] = acc


def _final(pall, wp, bp3):
    return pl.pallas_call(
        _final_body,
        grid=(1,),
        in_specs=[
            pl.BlockSpec((8, D), lambda i: (0, 0)),
            pl.BlockSpec((L + 1, D, OUT), lambda i: (0, 0, 0)),
            pl.BlockSpec((L + 1, 1, OUT), lambda i: (0, 0, 0)),
        ],
        out_specs=pl.BlockSpec((1, OUT), lambda i: (0, 0)),
        out_shape=jax.ShapeDtypeStruct((1, OUT), jnp.float32),
    )(pall, wp, bp3)


def kernel(x, edge_index, rel_type, basis, coef, W1, bn1_g, bn1_b,
           W2, bn2_g, bn2_b, Wp, bp):
    src = edge_index[0].astype(jnp.int32)
    dst = edge_index[1].astype(jnp.int32)
    rel = rel_type.astype(jnp.int32)

    pad = EPT_PAD - EPT
    gidx = rel * N + src
    gidx_t = jnp.pad(gidx.reshape(NTILE, EPT), ((0, 0), (0, pad)),
                     constant_values=0).reshape(NTILE, NCHUNK, CH)
    dst_t = jnp.pad(dst.reshape(NTILE, EPT), ((0, 0), (0, pad)),
                    constant_values=N).reshape(NTILE, NCHUNK, CH)
    edata = jnp.stack([gidx_t, dst_t], axis=2)
    zrows = jnp.zeros((ROWS_LAST, HALF), jnp.float32)

    pools = [_pool(x)]
    h = x
    for l in range(L):
        wfull = _wfull(coef[l].reshape(R, 1, NB), basis[l])
        hw0, hw1 = _hw(h, wfull)
        agg = _sc_aggregate(hw0.reshape(RN, HALF), hw1.reshape(RN, HALF),
                            zrows, edata)
        t, st1 = _mlp1(h, agg, W1[l])
        u, st2 = _bn_mm_stats(t, st1, bn1_g[l].reshape(1, D),
                              bn1_b[l].reshape(1, D), W2[l])
        h, ph = _bn_final(u, st2, bn2_g[l].reshape(1, D),
                          bn2_b[l].reshape(1, D))
        pools.append(ph)

    pall = jnp.concatenate([p[0:1] for p in pools]
                           + [jnp.zeros((8 - (L + 1), D), jnp.float32)],
                           axis=0)
    return _final(pall, Wp, bp.reshape(L + 1, 1, OUT))


# speedup vs baseline: 1.0759x; 1.0582x over previous
import jax
import jax.numpy as jnp
from jax import lax
from jax.experimental import pallas as pl
from jax.experimental.pallas import tpu as pltpu
from jax.experimental.pallas import tpu_sc as plsc

N = 10000
E = 160000
D = 256
R = 8
NB = 4
L = 4
OUT = 128
RN = R * N

NSC = 2
NTILE = 16
LANE = 16
HALF = D // NSC
EPT = E // NTILE
CH = 128
NCHUNK = 80
NPAIR = NCHUNK // 2
EPT_PAD = NCHUNK * CH
NACC = N + LANE
ROWS_T = 624
ROWS_LAST = N - (NTILE - 1) * ROWS_T
BN_ROWS = 1000
NI = N // BN_ROWS


def _sc_body(hw_hbm, zrows_hbm, ed_hbm, out_hbm,
             eb0, eb1, gbuf0, gbuf1, acc, gsem0, gsem1):
    sc = lax.axis_index("c")
    t = lax.axis_index("s")
    base = t * ROWS_T

    @pl.when(t < NTILE - 1)
    def _():
        pltpu.sync_copy(zrows_hbm.at[pl.ds(0, ROWS_T)],
                        acc.at[pl.ds(base, ROWS_T)])

    @pl.when(t == NTILE - 1)
    def _():
        pltpu.sync_copy(zrows_hbm, acc.at[pl.ds(base, ROWS_LAST)])

    plsc.subcore_barrier()

    goff = sc * RN

    def _bld(eb):
        def _bi(c, carry):
            sl = pl.ds(c * LANE, LANE)
            eb[0, sl] = eb[0, sl] + goff
            return carry
        lax.fori_loop(0, CH // LANE, _bi, 0)

    pltpu.sync_copy(ed_hbm.at[t, 0], eb0)
    _bld(eb0)
    pltpu.async_copy(hw_hbm.at[eb0.at[0]], gbuf0, gsem0)

    def _pair(j2, carry):
        j = 2 * j2
        pltpu.sync_copy(ed_hbm.at[t, j + 1], eb1)
        _bld(eb1)
        pltpu.async_copy(hw_hbm.at[eb1.at[0]], gbuf1, gsem1)
        pltpu.make_async_copy(hw_hbm.at[eb0.at[0]], gbuf0, gsem0).wait()
        pltpu.sync_copy(gbuf0, acc.at[eb0.at[1]], add=True)

        @pl.when(j2 < NPAIR - 1)
        def _():
            pltpu.sync_copy(ed_hbm.at[t, j + 2], eb0)
            _bld(eb0)
            pltpu.async_copy(hw_hbm.at[eb0.at[0]], gbuf0, gsem0)

        pltpu.make_async_copy(hw_hbm.at[eb1.at[0]], gbuf1, gsem1).wait()
        pltpu.sync_copy(gbuf1, acc.at[eb1.at[1]], add=True)
        return carry

    lax.fori_loop(0, NPAIR, _pair, 0)
    plsc.subcore_barrier()

    csl = pl.ds(sc * HALF, HALF)

    @pl.when(t < NTILE - 1)
    def _():
        pltpu.sync_copy(acc.at[pl.ds(base, ROWS_T)],
                        out_hbm.at[pl.ds(base, ROWS_T), csl])

    @pl.when(t == NTILE - 1)
    def _():
        pltpu.sync_copy(acc.at[pl.ds(base, ROWS_LAST)],
                        out_hbm.at[pl.ds(base, ROWS_LAST), csl])


def _sc_aggregate(hw_flat, zrows, edata):
    mesh = plsc.VectorSubcoreMesh(core_axis_name="c", subcore_axis_name="s")
    f = pl.kernel(
        _sc_body,
        out_type=jax.ShapeDtypeStruct((N, D), jnp.float32),
        mesh=mesh,
        scratch_types=[
            pltpu.VMEM((2, CH), jnp.int32),
            pltpu.VMEM((2, CH), jnp.int32),
            pltpu.VMEM((CH, HALF), jnp.float32),
            pltpu.VMEM((CH, HALF), jnp.float32),
            pltpu.VMEM_SHARED((NACC, HALF), jnp.float32),
            pltpu.SemaphoreType.DMA,
            pltpu.SemaphoreType.DMA,
        ],
    )
    return f(hw_flat, zrows, edata)


def _wfull_body(coef_ref, basis_ref, out_ref):
    acc = jnp.zeros((D, D), jnp.float32)
    for b in range(NB):
        acc = acc + coef_ref[0, 0, b] * basis_ref[b]
    out_ref[0] = acc


def _wfull(coef_l3, basis_l):
    return pl.pallas_call(
        _wfull_body,
        grid=(R,),
        in_specs=[
            pl.BlockSpec((1, 1, NB), lambda r: (r, 0, 0)),
            pl.BlockSpec((NB, D, D), lambda r: (0, 0, 0)),
        ],
        out_specs=pl.BlockSpec((1, D, D), lambda r: (r, 0, 0)),
        out_shape=jax.ShapeDtypeStruct((R, D, D), jnp.float32),
    )(coef_l3, basis_l)


def _hw_body(h_ref, w_ref, out_ref):
    out_ref[0, 0] = jnp.dot(h_ref[...], w_ref[0],
                            preferred_element_type=jnp.float32)


def _hw(h, wfull):
    return pl.pallas_call(
        _hw_body,
        grid=(NI, R, NSC),
        in_specs=[
            pl.BlockSpec((BN_ROWS, D), lambda ii, r, c: (ii, 0)),
            pl.BlockSpec((1, D, HALF), lambda ii, r, c: (r, 0, c)),
        ],
        out_specs=pl.BlockSpec((1, 1, BN_ROWS, HALF),
                               lambda ii, r, c: (c, r, ii, 0)),
        out_shape=jax.ShapeDtypeStruct((NSC, R, N, HALF), jnp.float32),
    )(h, wfull)


def _mlp1_body(h_ref, agg_ref, w_ref, y_ref, st_ref):
    i = pl.program_id(0)
    z = h_ref[...] + agg_ref[...]
    y = jnp.dot(z, w_ref[...], preferred_element_type=jnp.float32)
    y_ref[...] = y
    s1 = jnp.sum(y, axis=0, keepdims=True)
    s2 = jnp.sum(y * y, axis=0, keepdims=True)
    st = jnp.concatenate([s1, s2, jnp.zeros((6, D), jnp.float32)], axis=0)

    @pl.when(i == 0)
    def _():
        st_ref[...] = jnp.zeros_like(st_ref)

    st_ref[...] = st_ref[...] + st


def _mlp1(h, agg, w):
    return pl.pallas_call(
        _mlp1_body,
        grid=(NI,),
        in_specs=[
            pl.BlockSpec((BN_ROWS, D), lambda i: (i, 0)),
            pl.BlockSpec((BN_ROWS, D), lambda i: (i, 0)),
            pl.BlockSpec((D, D), lambda i: (0, 0)),
        ],
        out_specs=[
            pl.BlockSpec((BN_ROWS, D), lambda i: (i, 0)),
            pl.BlockSpec((8, D), lambda i: (0, 0)),
        ],
        out_shape=[
            jax.ShapeDtypeStruct((N, D), jnp.float32),
            jax.ShapeDtypeStruct((8, D), jnp.float32),
        ],
    )(h, agg, w)


def _bn_mm_stats_body(t_ref, st_in_ref, g_ref, b_ref, w_ref, y_ref, st_ref):
    i = pl.program_id(0)
    mean = st_in_ref[0:1, :] * (1.0 / N)
    ex2 = st_in_ref[1:2, :] * (1.0 / N)
    var = ex2 - mean * mean
    inv = g_ref[...] * lax.rsqrt(var + 1e-5)
    a = jnp.maximum((t_ref[...] - mean) * inv + b_ref[...], 0.0)
    y = jnp.dot(a, w_ref[...], preferred_element_type=jnp.float32)
    y_ref[...] = y
    s1 = jnp.sum(y, axis=0, keepdims=True)
    s2 = jnp.sum(y * y, axis=0, keepdims=True)
    st = jnp.concatenate([s1, s2, jnp.zeros((6, D), jnp.float32)], axis=0)

    @pl.when(i == 0)
    def _():
        st_ref[...] = jnp.zeros_like(st_ref)

    st_ref[...] = st_ref[...] + st


def _bn_mm_stats(t, st1, g, b, w):
    return pl.pallas_call(
        _bn_mm_stats_body,
        grid=(NI,),
        in_specs=[
            pl.BlockSpec((BN_ROWS, D), lambda i: (i, 0)),
            pl.BlockSpec((8, D), lambda i: (0, 0)),
            pl.BlockSpec((1, D), lambda i: (0, 0)),
            pl.BlockSpec((1, D), lambda i: (0, 0)),
            pl.BlockSpec((D, D), lambda i: (0, 0)),
        ],
        out_specs=[
            pl.BlockSpec((BN_ROWS, D), lambda i: (i, 0)),
            pl.BlockSpec((8, D), lambda i: (0, 0)),
        ],
        out_shape=[
            jax.ShapeDtypeStruct((N, D), jnp.float32),
            jax.ShapeDtypeStruct((8, D), jnp.float32),
        ],
    )(t, st1, g, b, w)


def _bn_final_body(u_ref, st_ref, g_ref, b_ref, h_ref, p_ref):
    ii = pl.program_id(0)
    mean = st_ref[0:1, :] * (1.0 / N)
    ex2 = st_ref[1:2, :] * (1.0 / N)
    var = ex2 - mean * mean
    inv = g_ref[...] * lax.rsqrt(var + 1e-5)
    h = jnp.maximum((u_ref[...] - mean) * inv + b_ref[...], 0.0)
    h_ref[...] = h
    ps = jnp.concatenate(
        [jnp.sum(h, axis=0, keepdims=True), jnp.zeros((7, D), jnp.float32)],
        axis=0)

    @pl.when(ii == 0)
    def _():
        p_ref[...] = jnp.zeros_like(p_ref)

    p_ref[...] = p_ref[...] + ps


def _bn_final(u, st2, g, b):
    return pl.pallas_call(
        _bn_final_body,
        grid=(NI,),
        in_specs=[
            pl.BlockSpec((BN_ROWS, D), lambda ii: (ii, 0)),
            pl.BlockSpec((8, D), lambda ii: (0, 0)),
            pl.BlockSpec((1, D), lambda ii: (0, 0)),
            pl.BlockSpec((1, D), lambda ii: (0, 0)),
        ],
        out_specs=[
            pl.BlockSpec((BN_ROWS, D), lambda ii: (ii, 0)),
            pl.BlockSpec((8, D), lambda ii: (0, 0)),
        ],
        out_shape=[
            jax.ShapeDtypeStruct((N, D), jnp.float32),
            jax.ShapeDtypeStruct((8, D), jnp.float32),
        ],
    )(u, st2, g, b)


def _pool_body(x_ref, p_ref):
    ii = pl.program_id(0)
    ps = jnp.concatenate(
        [jnp.sum(x_ref[...], axis=0, keepdims=True),
         jnp.zeros((7, D), jnp.float32)], axis=0)

    @pl.when(ii == 0)
    def _():
        p_ref[...] = jnp.zeros_like(p_ref)

    p_ref[...] = p_ref[...] + ps


def _pool(x):
    return pl.pallas_call(
        _pool_body,
        grid=(NI,),
        in_specs=[pl.BlockSpec((BN_ROWS, D), lambda ii: (ii, 0))],
        out_specs=pl.BlockSpec((8, D), lambda ii: (0, 0)),
        out_shape=jax.ShapeDtypeStruct((8, D), jnp.float32),
    )(x)


def _final_body(p_ref, wp_ref, bp_ref, out_ref):
    acc = jnp.zeros((1, OUT), jnp.float32)
    for i in range(L + 1):
        acc = acc + jnp.dot(p_ref[i:i + 1, :], wp_ref[i],
                            preferred_element_type=jnp.float32)
    acc = acc + jnp.sum(bp_ref[:, 0, :], axis=0, keepdims=True)
    out_ref[...] = acc


def _final(pall, wp, bp3):
    return pl.pallas_call(
        _final_body,
        grid=(1,),
        in_specs=[
            pl.BlockSpec((8, D), lambda i: (0, 0)),
            pl.BlockSpec((L + 1, D, OUT), lambda i: (0, 0, 0)),
            pl.BlockSpec((L + 1, 1, OUT), lambda i: (0, 0, 0)),
        ],
        out_specs=pl.BlockSpec((1, OUT), lambda i: (0, 0)),
        out_shape=jax.ShapeDtypeStruct((1, OUT), jnp.float32),
    )(pall, wp, bp3)


def kernel(x, edge_index, rel_type, basis, coef, W1, bn1_g, bn1_b,
           W2, bn2_g, bn2_b, Wp, bp):
    src = edge_index[0].astype(jnp.int32)
    dst = edge_index[1].astype(jnp.int32)
    rel = rel_type.astype(jnp.int32)

    pad = EPT_PAD - EPT
    gidx = rel * N + src
    gidx_t = jnp.pad(gidx.reshape(NTILE, EPT), ((0, 0), (0, pad)),
                     constant_values=0).reshape(NTILE, NCHUNK, CH)
    dst_t = jnp.pad(dst.reshape(NTILE, EPT), ((0, 0), (0, pad)),
                    constant_values=N).reshape(NTILE, NCHUNK, CH)
    edata = jnp.stack([gidx_t, dst_t], axis=2)
    zrows = jnp.zeros((ROWS_LAST, HALF), jnp.float32)

    pools = [_pool(x)]
    h = x
    for l in range(L):
        wfull = _wfull(coef[l].reshape(R, 1, NB), basis[l])
        hw = _hw(h, wfull)
        agg = _sc_aggregate(hw.reshape(NSC * RN, HALF), zrows, edata)
        t, st1 = _mlp1(h, agg, W1[l])
        u, st2 = _bn_mm_stats(t, st1, bn1_g[l].reshape(1, D),
                              bn1_b[l].reshape(1, D), W2[l])
        h, ph = _bn_final(u, st2, bn2_g[l].reshape(1, D),
                          bn2_b[l].reshape(1, D))
        pools.append(ph)

    pall = jnp.concatenate([p[0:1] for p in pools]
                           + [jnp.zeros((8 - (L + 1), D), jnp.float32)],
                           axis=0)
    return _final(pall, Wp, bp.reshape(L + 1, 1, OUT))


# speedup vs baseline: 1.2042x; 1.1193x over previous
import jax
import jax.numpy as jnp
from jax import lax
from jax.experimental import pallas as pl
from jax.experimental.pallas import tpu as pltpu
from jax.experimental.pallas import tpu_sc as plsc

N = 10000
E = 160000
D = 256
R = 8
NB = 4
L = 4
OUT = 128
RN = R * N

NSC = 2
NTILE = 16
LANE = 16
HALF = D // NSC
EPT = E // NTILE
CH = 128
NCHUNK = 80
NPAIR = NCHUNK // 2
EPT_PAD = NCHUNK * CH
NACC = N + LANE
ROWS_T = 624
ROWS_LAST = N - (NTILE - 1) * ROWS_T
BN_ROWS = 1000
NI = N // BN_ROWS


def _sc_body(hw0_hbm, hw1_hbm, zrows_hbm, ed_hbm, out_hbm,
             eb0, eb1, gbuf0, gbuf1, acc, gsem0, gsem1):
    sc = lax.axis_index("c")
    t = lax.axis_index("s")
    base = t * ROWS_T

    @pl.when(t < NTILE - 1)
    def _():
        pltpu.sync_copy(zrows_hbm.at[pl.ds(0, ROWS_T)],
                        acc.at[pl.ds(base, ROWS_T)])

    @pl.when(t == NTILE - 1)
    def _():
        pltpu.sync_copy(zrows_hbm, acc.at[pl.ds(base, ROWS_LAST)])

    plsc.subcore_barrier()

    def _run(tab):
        pltpu.sync_copy(ed_hbm.at[t, 0], eb0)
        pltpu.async_copy(tab.at[eb0.at[0]], gbuf0, gsem0)

        def _pair(j2, carry):
            j = 2 * j2
            pltpu.sync_copy(ed_hbm.at[t, j + 1], eb1)
            pltpu.async_copy(tab.at[eb1.at[0]], gbuf1, gsem1)
            pltpu.make_async_copy(tab.at[eb0.at[0]], gbuf0, gsem0).wait()
            pltpu.sync_copy(gbuf0, acc.at[eb0.at[1]], add=True)

            @pl.when(j2 < NPAIR - 1)
            def _():
                pltpu.sync_copy(ed_hbm.at[t, j + 2], eb0)
                pltpu.async_copy(tab.at[eb0.at[0]], gbuf0, gsem0)

            pltpu.make_async_copy(tab.at[eb1.at[0]], gbuf1, gsem1).wait()
            pltpu.sync_copy(gbuf1, acc.at[eb1.at[1]], add=True)
            return carry

        lax.fori_loop(0, NPAIR, _pair, 0)

    @pl.when(sc == 0)
    def _():
        _run(hw0_hbm)

    @pl.when(sc == 1)
    def _():
        _run(hw1_hbm)

    plsc.subcore_barrier()

    csl = pl.ds(sc * HALF, HALF)

    @pl.when(t < NTILE - 1)
    def _():
        pltpu.sync_copy(acc.at[pl.ds(base, ROWS_T)],
                        out_hbm.at[pl.ds(base, ROWS_T), csl])

    @pl.when(t == NTILE - 1)
    def _():
        pltpu.sync_copy(acc.at[pl.ds(base, ROWS_LAST)],
                        out_hbm.at[pl.ds(base, ROWS_LAST), csl])


def _sc_aggregate(hw0, hw1, zrows, edata):
    mesh = plsc.VectorSubcoreMesh(core_axis_name="c", subcore_axis_name="s")
    f = pl.kernel(
        _sc_body,
        out_type=jax.ShapeDtypeStruct((N, D), jnp.float32),
        mesh=mesh,
        scratch_types=[
            pltpu.VMEM((2, CH), jnp.int32),
            pltpu.VMEM((2, CH), jnp.int32),
            pltpu.VMEM((CH, HALF), jnp.float32),
            pltpu.VMEM((CH, HALF), jnp.float32),
            pltpu.VMEM_SHARED((NACC, HALF), jnp.float32),
            pltpu.SemaphoreType.DMA,
            pltpu.SemaphoreType.DMA,
        ],
    )
    return f(hw0, hw1, zrows, edata)


def _wfull_body(coef_ref, basis_ref, out_ref):
    acc = jnp.zeros((D, D), jnp.float32)
    for b in range(NB):
        acc = acc + coef_ref[0, 0, b] * basis_ref[b]
    out_ref[0] = acc


def _wfull(coef_l3, basis_l):
    return pl.pallas_call(
        _wfull_body,
        grid=(R,),
        in_specs=[
            pl.BlockSpec((1, 1, NB), lambda r: (r, 0, 0)),
            pl.BlockSpec((NB, D, D), lambda r: (0, 0, 0)),
        ],
        out_specs=pl.BlockSpec((1, D, D), lambda r: (r, 0, 0)),
        out_shape=jax.ShapeDtypeStruct((R, D, D), jnp.float32),
    )(coef_l3, basis_l)


def _hw_body(h_ref, w_ref, out0_ref, out1_ref):
    y = jnp.dot(h_ref[...], w_ref[0], preferred_element_type=jnp.float32)
    out0_ref[0] = y[:, :HALF]
    out1_ref[0] = y[:, HALF:]


def _hw(h, wfull):
    return pl.pallas_call(
        _hw_body,
        grid=(NI, R),
        in_specs=[
            pl.BlockSpec((BN_ROWS, D), lambda ii, r: (ii, 0)),
            pl.BlockSpec((1, D, D), lambda ii, r: (r, 0, 0)),
        ],
        out_specs=[
            pl.BlockSpec((1, BN_ROWS, HALF), lambda ii, r: (r, ii, 0)),
            pl.BlockSpec((1, BN_ROWS, HALF), lambda ii, r: (r, ii, 0)),
        ],
        out_shape=[
            jax.ShapeDtypeStruct((R, N, HALF), jnp.float32),
            jax.ShapeDtypeStruct((R, N, HALF), jnp.float32),
        ],
    )(h, wfull)


def _mlp1_body(h_ref, agg_ref, w_ref, y_ref, st_ref):
    i = pl.program_id(0)
    z = h_ref[...] + agg_ref[...]
    y = jnp.dot(z, w_ref[...], preferred_element_type=jnp.float32)
    y_ref[...] = y
    s1 = jnp.sum(y, axis=0, keepdims=True)
    s2 = jnp.sum(y * y, axis=0, keepdims=True)
    st = jnp.concatenate([s1, s2, jnp.zeros((6, D), jnp.float32)], axis=0)

    @pl.when(i == 0)
    def _():
        st_ref[...] = jnp.zeros_like(st_ref)

    st_ref[...] = st_ref[...] + st


def _mlp1(h, agg, w):
    return pl.pallas_call(
        _mlp1_body,
        grid=(NI,),
        in_specs=[
            pl.BlockSpec((BN_ROWS, D), lambda i: (i, 0)),
            pl.BlockSpec((BN_ROWS, D), lambda i: (i, 0)),
            pl.BlockSpec((D, D), lambda i: (0, 0)),
        ],
        out_specs=[
            pl.BlockSpec((BN_ROWS, D), lambda i: (i, 0)),
            pl.BlockSpec((8, D), lambda i: (0, 0)),
        ],
        out_shape=[
            jax.ShapeDtypeStruct((N, D), jnp.float32),
            jax.ShapeDtypeStruct((8, D), jnp.float32),
        ],
    )(h, agg, w)


def _bn_mm_stats_body(t_ref, st_in_ref, g_ref, b_ref, w_ref, y_ref, st_ref):
    i = pl.program_id(0)
    mean = st_in_ref[0:1, :] * (1.0 / N)
    ex2 = st_in_ref[1:2, :] * (1.0 / N)
    var = ex2 - mean * mean
    inv = g_ref[...] * lax.rsqrt(var + 1e-5)
    a = jnp.maximum((t_ref[...] - mean) * inv + b_ref[...], 0.0)
    y = jnp.dot(a, w_ref[...], preferred_element_type=jnp.float32)
    y_ref[...] = y
    s1 = jnp.sum(y, axis=0, keepdims=True)
    s2 = jnp.sum(y * y, axis=0, keepdims=True)
    st = jnp.concatenate([s1, s2, jnp.zeros((6, D), jnp.float32)], axis=0)

    @pl.when(i == 0)
    def _():
        st_ref[...] = jnp.zeros_like(st_ref)

    st_ref[...] = st_ref[...] + st


def _bn_mm_stats(t, st1, g, b, w):
    return pl.pallas_call(
        _bn_mm_stats_body,
        grid=(NI,),
        in_specs=[
            pl.BlockSpec((BN_ROWS, D), lambda i: (i, 0)),
            pl.BlockSpec((8, D), lambda i: (0, 0)),
            pl.BlockSpec((1, D), lambda i: (0, 0)),
            pl.BlockSpec((1, D), lambda i: (0, 0)),
            pl.BlockSpec((D, D), lambda i: (0, 0)),
        ],
        out_specs=[
            pl.BlockSpec((BN_ROWS, D), lambda i: (i, 0)),
            pl.BlockSpec((8, D), lambda i: (0, 0)),
        ],
        out_shape=[
            jax.ShapeDtypeStruct((N, D), jnp.float32),
            jax.ShapeDtypeStruct((8, D), jnp.float32),
        ],
    )(t, st1, g, b, w)


def _bn_final_body(u_ref, st_ref, g_ref, b_ref, h_ref, p_ref):
    ii = pl.program_id(0)
    mean = st_ref[0:1, :] * (1.0 / N)
    ex2 = st_ref[1:2, :] * (1.0 / N)
    var = ex2 - mean * mean
    inv = g_ref[...] * lax.rsqrt(var + 1e-5)
    h = jnp.maximum((u_ref[...] - mean) * inv + b_ref[...], 0.0)
    h_ref[...] = h
    ps = jnp.concatenate(
        [jnp.sum(h, axis=0, keepdims=True), jnp.zeros((7, D), jnp.float32)],
        axis=0)

    @pl.when(ii == 0)
    def _():
        p_ref[...] = jnp.zeros_like(p_ref)

    p_ref[...] = p_ref[...] + ps


def _bn_final(u, st2, g, b):
    return pl.pallas_call(
        _bn_final_body,
        grid=(NI,),
        in_specs=[
            pl.BlockSpec((BN_ROWS, D), lambda ii: (ii, 0)),
            pl.BlockSpec((8, D), lambda ii: (0, 0)),
            pl.BlockSpec((1, D), lambda ii: (0, 0)),
            pl.BlockSpec((1, D), lambda ii: (0, 0)),
        ],
        out_specs=[
            pl.BlockSpec((BN_ROWS, D), lambda ii: (ii, 0)),
            pl.BlockSpec((8, D), lambda ii: (0, 0)),
        ],
        out_shape=[
            jax.ShapeDtypeStruct((N, D), jnp.float32),
            jax.ShapeDtypeStruct((8, D), jnp.float32),
        ],
    )(u, st2, g, b)


def _pool_body(x_ref, p_ref):
    ii = pl.program_id(0)
    ps = jnp.concatenate(
        [jnp.sum(x_ref[...], axis=0, keepdims=True),
         jnp.zeros((7, D), jnp.float32)], axis=0)

    @pl.when(ii == 0)
    def _():
        p_ref[...] = jnp.zeros_like(p_ref)

    p_ref[...] = p_ref[...] + ps


def _pool(x):
    return pl.pallas_call(
        _pool_body,
        grid=(NI,),
        in_specs=[pl.BlockSpec((BN_ROWS, D), lambda ii: (ii, 0))],
        out_specs=pl.BlockSpec((8, D), lambda ii: (0, 0)),
        out_shape=jax.ShapeDtypeStruct((8, D), jnp.float32),
    )(x)


def _final_body(p_ref, wp_ref, bp_ref, out_ref):
    acc = jnp.zeros((1, OUT), jnp.float32)
    for i in range(L + 1):
        acc = acc + jnp.dot(p_ref[i:i + 1, :], wp_ref[i],
                            preferred_element_type=jnp.float32)
    acc = acc + jnp.sum(bp_ref[:, 0, :], axis=0, keepdims=True)
    out_ref[...] = acc


def _final(pall, wp, bp3):
    return pl.pallas_call(
        _final_body,
        grid=(1,),
        in_specs=[
            pl.BlockSpec((8, D), lambda i: (0, 0)),
            pl.BlockSpec((L + 1, D, OUT), lambda i: (0, 0, 0)),
            pl.BlockSpec((L + 1, 1, OUT), lambda i: (0, 0, 0)),
        ],
        out_specs=pl.BlockSpec((1, OUT), lambda i: (0, 0)),
        out_shape=jax.ShapeDtypeStruct((1, OUT), jnp.float32),
    )(pall, wp, bp3)


def kernel(x, edge_index, rel_type, basis, coef, W1, bn1_g, bn1_b,
           W2, bn2_g, bn2_b, Wp, bp):
    src = edge_index[0].astype(jnp.int32)
    dst = edge_index[1].astype(jnp.int32)
    rel = rel_type.astype(jnp.int32)

    pad = EPT_PAD - EPT
    gidx = rel * N + src
    gidx_t = jnp.pad(gidx.reshape(NTILE, EPT), ((0, 0), (0, pad)),
                     constant_values=0).reshape(NTILE, NCHUNK, CH)
    dst_t = jnp.pad(dst.reshape(NTILE, EPT), ((0, 0), (0, pad)),
                    constant_values=N).reshape(NTILE, NCHUNK, CH)
    edata = jnp.stack([gidx_t, dst_t], axis=2)
    zrows = jnp.zeros((ROWS_LAST, HALF), jnp.float32)

    pools = [_pool(x)]
    h = x
    for l in range(L):
        wfull = _wfull(coef[l].reshape(R, 1, NB), basis[l])
        hw0, hw1 = _hw(h, wfull)
        agg = _sc_aggregate(hw0.reshape(RN, HALF), hw1.reshape(RN, HALF),
                            zrows, edata)
        t, st1 = _mlp1(h, agg, W1[l])
        u, st2 = _bn_mm_stats(t, st1, bn1_g[l].reshape(1, D),
                              bn1_b[l].reshape(1, D), W2[l])
        h, ph = _bn_final(u, st2, bn2_g[l].reshape(1, D),
                          bn2_b[l].reshape(1, D))
        pools.append(ph)

    pall = jnp.concatenate([p[0:1] for p in pools]
                           + [jnp.zeros((8 - (L + 1), D), jnp.float32)],
                           axis=0)
    return _final(pall, Wp, bp.reshape(L + 1, 1, OUT))
